# baseline (speedup 1.0000x reference)
import functools

import jax
import jax.numpy as jnp
from jax import lax
from jax.experimental import pallas as pl
from jax.experimental.pallas import tpu as pltpu
from jax.experimental.pallas import tpu_sc as plsc

N_NODES = 10000
N_PAD = 10240
HID = 128
RBF = 6
RBF_PAD = 8
NODE_BLK = 1000
EDGE_BLK = 1024
E_PAD = 327680


def _silu(x):
    return x * jax.nn.sigmoid(x)


def _dot(a, b):
    return jnp.dot(a, b, preferred_element_type=jnp.float32)


def _in_mlp_body(x_ref, w_ref, b_ref, o_ref):
    o_ref[...] = _silu(_dot(x_ref[...], w_ref[...]) + b_ref[...])


def _precompute_body(h_ref, ws_ref, wd_ref, b1_ref, a_ref, b_ref):
    h = h_ref[...]
    a_ref[...] = _dot(h, ws_ref[...]) + b1_ref[...]
    b_ref[...] = _dot(h, wd_ref[...])


def _edge_body(ga_ref, gb_ref, rbf_ref, wr_ref, w2_ref, b2_ref, m_ref):
    g = ga_ref[...] + gb_ref[...] + _dot(rbf_ref[...], wr_ref[...])
    m = _silu(g)
    m = _silu(_dot(m, w2_ref[...]) + b2_ref[...])
    m_ref[...] = m * jax.nn.sigmoid(m)


def _update_body(s0_ref, s1_ref, h_ref, u1_ref, c1_ref, u2_ref, c2_ref, o_ref):
    h = h_ref[...]
    u = s0_ref[...] + s1_ref[...] + h
    o_ref[...] = _dot(_silu(_dot(u, u1_ref[...]) + c1_ref[...]), u2_ref[...]) \
        + c2_ref[...] + h


def _readout_body(h_ref, w1_ref, b1_ref, w2_ref, b2_ref,
                  ow1_ref, ob1_ref, ow2t_ref, ob2_ref, o_ref, sacc, macc):
    i = pl.program_id(0)
    y = _dot(_silu(_dot(h_ref[...], w1_ref[...]) + b1_ref[...]), w2_ref[...]) \
        + b2_ref[...]
    s = jnp.sum(y, axis=0, keepdims=True)
    m = jnp.max(y, axis=0, keepdims=True)

    @pl.when(i == 0)
    def _():
        sacc[...] = s
        macc[...] = m

    @pl.when(i > 0)
    def _():
        sacc[...] = sacc[...] + s
        macc[...] = jnp.maximum(macc[...], m)

    @pl.when(i == pl.num_programs(0) - 1)
    def _():
        t = _dot(sacc[...], ow1_ref[0:HID, :]) \
            + _dot(macc[...], ow1_ref[HID:2 * HID, :]) + ob1_ref[...]
        o_ref[...] = jnp.sum(_silu(t) * ow2t_ref[...], axis=1, keepdims=True) \
            + ob2_ref[...]


def _node_spec():
    return pl.BlockSpec((NODE_BLK, HID), lambda i: (i, 0))


def _w_spec(r, c):
    return pl.BlockSpec((r, c), lambda i: (0, 0))


def _input_mlp(x, W_in, b_in):
    return pl.pallas_call(
        _in_mlp_body,
        grid=(N_NODES // NODE_BLK,),
        in_specs=[_node_spec(), _w_spec(HID, HID), _w_spec(1, HID)],
        out_specs=_node_spec(),
        out_shape=jax.ShapeDtypeStruct((N_NODES, HID), jnp.float32),
    )(x, W_in, b_in.reshape(1, HID))


def _precompute(h, w1s, w1d, b1):
    return pl.pallas_call(
        _precompute_body,
        grid=(N_NODES // NODE_BLK,),
        in_specs=[_node_spec(), _w_spec(HID, HID), _w_spec(HID, HID),
                  _w_spec(1, HID)],
        out_specs=(_node_spec(), _node_spec()),
        out_shape=(jax.ShapeDtypeStruct((N_NODES, HID), jnp.float32),
                   jax.ShapeDtypeStruct((N_NODES, HID), jnp.float32)),
    )(h, w1s, w1d, b1.reshape(1, HID))


def _edge_mlp(ga, gb, rbf_p, w1r, w2, b2):
    eblk = pl.BlockSpec((EDGE_BLK, HID), lambda i: (i, 0))
    return pl.pallas_call(
        _edge_body,
        grid=(E_PAD // EDGE_BLK,),
        in_specs=[eblk, eblk,
                  pl.BlockSpec((EDGE_BLK, RBF_PAD), lambda i: (i, 0)),
                  _w_spec(RBF_PAD, HID), _w_spec(HID, HID), _w_spec(1, HID)],
        out_specs=eblk,
        out_shape=jax.ShapeDtypeStruct((E_PAD, HID), jnp.float32),
    )(ga, gb, rbf_p, w1r, w2, b2.reshape(1, HID))


def _update(s0, s1, h, u1, c1, u2, c2):
    return pl.pallas_call(
        _update_body,
        grid=(N_NODES // NODE_BLK,),
        in_specs=[_node_spec(), _node_spec(), _node_spec(),
                  _w_spec(HID, HID), _w_spec(1, HID),
                  _w_spec(HID, HID), _w_spec(1, HID)],
        out_specs=_node_spec(),
        out_shape=jax.ShapeDtypeStruct((N_NODES, HID), jnp.float32),
    )(s0, s1, h, u1, c1.reshape(1, HID), u2, c2.reshape(1, HID))


def _readout(h, won_W1, won_b1, won_W2, won_b2, out_W1, out_b1, out_W2, out_b2):
    return pl.pallas_call(
        _readout_body,
        grid=(N_NODES // NODE_BLK,),
        in_specs=[_node_spec(), _w_spec(HID, HID), _w_spec(1, HID),
                  _w_spec(HID, HID), _w_spec(1, HID),
                  _w_spec(2 * HID, HID), _w_spec(1, HID),
                  _w_spec(1, HID), _w_spec(1, 1)],
        out_specs=pl.BlockSpec((1, 1), lambda i: (0, 0)),
        out_shape=jax.ShapeDtypeStruct((1, 1), jnp.float32),
        scratch_shapes=[pltpu.VMEM((1, HID), jnp.float32),
                        pltpu.VMEM((1, HID), jnp.float32)],
    )(h, won_W1, won_b1.reshape(1, HID), won_W2, won_b2.reshape(1, HID),
      out_W1, out_b1.reshape(1, HID), out_W2.T.reshape(1, HID),
      out_b2.reshape(1, 1))


SC_NC = 2
SC_NS = 16
NW = SC_NC * SC_NS
EPW = E_PAD // NW
CH = 128
NCH = EPW // CH
ROWS_PER_TILE = N_PAD // SC_NS

GNCH0 = 48
GNCH1 = 112
GNCHMAX = max(GNCH0, GNCH1)

_SC_MESH = plsc.VectorSubcoreMesh(core_axis_name="c", subcore_axis_name="s")


@functools.partial(
    pl.kernel,
    out_type=(jax.ShapeDtypeStruct((E_PAD, HID), jnp.float32),
              jax.ShapeDtypeStruct((E_PAD, HID), jnp.float32)),
    mesh=_SC_MESH,
    scratch_types=[
        pltpu.VMEM((GNCHMAX * CH,), jnp.int32),
        pltpu.VMEM((GNCHMAX * CH,), jnp.int32),
        pltpu.VMEM((CH, HID), jnp.float32),
        pltpu.VMEM((CH, HID), jnp.float32),
        pltpu.VMEM((CH, HID), jnp.float32),
        pltpu.VMEM((CH, HID), jnp.float32),
        pltpu.SemaphoreType.DMA,
        pltpu.SemaphoreType.DMA,
        pltpu.SemaphoreType.DMA,
        pltpu.SemaphoreType.DMA,
    ],
)
def _sc_gather(a_hbm, b_hbm, src_hbm, dst_hbm, ga_hbm, gb_hbm,
               sidx_v, didx_v, ra0_v, rb0_v, ra1_v, rb1_v,
               sa0, sb0, sa1, sb1):
    cid = lax.axis_index("c")
    sid = lax.axis_index("s")
    nch = jnp.where(cid == 0, GNCH0, GNCH1)
    base = jnp.where(cid == 0, sid * (GNCH0 * CH),
                     SC_NS * (GNCH0 * CH) + sid * (GNCH1 * CH))
    pltpu.sync_copy(src_hbm.at[pl.ds(base, GNCHMAX * CH)], sidx_v)
    pltpu.sync_copy(dst_hbm.at[pl.ds(base, GNCHMAX * CH)], didx_v)

    bufs = ((ra0_v, rb0_v, sa0, sb0), (ra1_v, rb1_v, sa1, sb1))

    def issue(chunk, ra, rb, sa, sb):
        off = chunk * CH
        pltpu.async_copy(a_hbm.at[sidx_v.at[pl.ds(off, CH)]], ra, sa)
        pltpu.async_copy(b_hbm.at[didx_v.at[pl.ds(off, CH)]], rb, sb)

    for par in (0, 1):
        issue(par, *bufs[par])

    def body(jj, carry):
        for par in (0, 1):
            ra, rb, sa, sb = bufs[par]
            i = jj * 2 + par
            off = i * CH
            pltpu.make_async_copy(
                a_hbm.at[sidx_v.at[pl.ds(off, CH)]], ra, sa).wait()
            pltpu.make_async_copy(
                b_hbm.at[didx_v.at[pl.ds(off, CH)]], rb, sb).wait()
            pltpu.sync_copy(ra, ga_hbm.at[pl.ds(base + off, CH)])
            pltpu.sync_copy(rb, gb_hbm.at[pl.ds(base + off, CH)])

            @pl.when(i + 2 < nch)
            def _():
                issue(i + 2, ra, rb, sa, sb)
        return carry

    lax.fori_loop(0, nch // 2, body, 0)


@functools.partial(
    pl.kernel,
    out_type=jax.ShapeDtypeStruct((SC_NC, N_PAD, HID), jnp.float32),
    mesh=_SC_MESH,
    scratch_types=[
        pltpu.VMEM((NCH, CH), jnp.int32),
        pltpu.VMEM((CH, HID), jnp.float32),
        pltpu.VMEM((CH, HID), jnp.float32),
        pltpu.VMEM_SHARED((N_PAD, HID), jnp.float32),
        pltpu.SemaphoreType.DMA,
        pltpu.SemaphoreType.DMA,
    ],
)
def _sc_scatter(m_hbm, dst3_hbm, zero_hbm, out_hbm,
                idx_v, r0_v, r1_v, acc_sh, sl0, sl1):
    cid = lax.axis_index("c")
    sid = lax.axis_index("s")
    wid = sid * SC_NC + cid
    base = wid * EPW
    pltpu.sync_copy(dst3_hbm.at[wid], idx_v)
    pltpu.async_copy(m_hbm.at[pl.ds(base, CH)], r0_v, sl0)
    pltpu.async_copy(m_hbm.at[pl.ds(base + CH, CH)], r1_v, sl1)
    pltpu.sync_copy(zero_hbm, acc_sh.at[pl.ds(sid * ROWS_PER_TILE,
                                              ROWS_PER_TILE)])
    plsc.subcore_barrier()

    bufs = ((r0_v, sl0), (r1_v, sl1))

    def body(jj, carry):
        for par in (0, 1):
            rv, sl = bufs[par]
            i = jj * 2 + par
            off = base + i * CH
            pltpu.make_async_copy(m_hbm.at[pl.ds(off, CH)], rv, sl).wait()
            pltpu.sync_copy(rv, acc_sh.at[idx_v.at[i]], add=True)

            @pl.when(i + 2 < NCH)
            def _():
                pltpu.async_copy(m_hbm.at[pl.ds(off + 2 * CH, CH)], rv, sl)
        return carry

    lax.fori_loop(0, NCH // 2, body, 0)
    plsc.subcore_barrier()
    r0 = sid * ROWS_PER_TILE
    pltpu.sync_copy(acc_sh.at[pl.ds(r0, ROWS_PER_TILE)],
                    out_hbm.at[cid, pl.ds(r0, ROWS_PER_TILE)])


def _gather_ab(a_tab, b_tab, src_p, dst_g):
    return _sc_gather(a_tab, b_tab, src_p, dst_g)


def _scatter_sum(m, dst_p, zero_rows):
    s = _sc_scatter(m, dst_p.reshape(NW, NCH, CH), zero_rows)
    return s[0, :N_NODES], s[1, :N_NODES]


def kernel(x, edge_index, d_rbf, W_in, b_in, msg_W1, msg_b1, msg_W2, msg_b2,
           upd_W1, upd_b1, upd_W2, upd_b2, won_W1, won_b1, won_W2, won_b2,
           out_W1, out_b1, out_W2, out_b2):
    E = edge_index.shape[1]
    pad = E_PAD - E
    src_p = jnp.concatenate([edge_index[0], jnp.zeros((pad,), jnp.int32)])
    dst_p = jnp.concatenate([edge_index[1],
                             jnp.full((pad,), N_NODES, jnp.int32)])
    dst_g = jnp.concatenate([edge_index[1], jnp.zeros((pad,), jnp.int32)])
    rbf_p = jnp.zeros((E_PAD, RBF_PAD), jnp.float32).at[:E, :RBF].set(d_rbf)
    zero_rows = jnp.zeros((ROWS_PER_TILE, HID), jnp.float32)

    h = _input_mlp(x, W_in, b_in)
    n_layers = msg_W1.shape[0]
    for l in range(n_layers):
        w1s = msg_W1[l, :HID]
        w1d = msg_W1[l, HID:2 * HID]
        w1r = jnp.zeros((RBF_PAD, HID), jnp.float32).at[:RBF].set(
            msg_W1[l, 2 * HID:])
        a_tab, b_tab = _precompute(h, w1s, w1d, msg_b1[l])
        ga, gb = _gather_ab(a_tab, b_tab, src_p, dst_g)
        m = _edge_mlp(ga, gb, rbf_p, w1r, msg_W2[l], msg_b2[l])
        s0, s1 = _scatter_sum(m, dst_p, zero_rows)
        h = _update(s0, s1, h, upd_W1[l], upd_b1[l], upd_W2[l], upd_b2[l])

    return _readout(h, won_W1, won_b1, won_W2, won_b2,
                    out_W1, out_b1, out_W2, out_b2)


# speedup vs baseline: 1.0220x; 1.0220x over previous
import functools

import jax
import jax.numpy as jnp
from jax import lax
from jax.experimental import pallas as pl
from jax.experimental.pallas import tpu as pltpu
from jax.experimental.pallas import tpu_sc as plsc

N_NODES = 10000
N_PAD = 10240
HID = 128
RBF = 6
RBF_PAD = 8
NODE_BLK = 1000
EDGE_BLK = 1024
E_PAD = 327680


def _silu(x):
    return x * jax.nn.sigmoid(x)


def _dot(a, b):
    return jnp.dot(a, b, preferred_element_type=jnp.float32)


def _in_mlp_body(x_ref, w_ref, b_ref, o_ref):
    o_ref[...] = _silu(_dot(x_ref[...], w_ref[...]) + b_ref[...])


def _precompute_body(h_ref, ws_ref, wd_ref, b1_ref, a_ref, b_ref):
    h = h_ref[...]
    a_ref[...] = _dot(h, ws_ref[...]) + b1_ref[...]
    b_ref[...] = _dot(h, wd_ref[...])


def _edge_body(ga_ref, gb_ref, rbf_ref, wr_ref, w2_ref, b2_ref, m_ref):
    g = ga_ref[...] + gb_ref[...] + _dot(rbf_ref[...], wr_ref[...])
    m = _silu(g)
    m = _silu(_dot(m, w2_ref[...]) + b2_ref[...])
    m_ref[...] = m * jax.nn.sigmoid(m)


def _update_body(s0_ref, s1_ref, h_ref, u1_ref, c1_ref, u2_ref, c2_ref, o_ref):
    h = h_ref[...]
    u = s0_ref[...] + s1_ref[...] + h
    o_ref[...] = _dot(_silu(_dot(u, u1_ref[...]) + c1_ref[...]), u2_ref[...]) \
        + c2_ref[...] + h


def _readout_body(h_ref, w1_ref, b1_ref, w2_ref, b2_ref,
                  ow1_ref, ob1_ref, ow2t_ref, ob2_ref, o_ref, sacc, macc):
    i = pl.program_id(0)
    y = _dot(_silu(_dot(h_ref[...], w1_ref[...]) + b1_ref[...]), w2_ref[...]) \
        + b2_ref[...]
    s = jnp.sum(y, axis=0, keepdims=True)
    m = jnp.max(y, axis=0, keepdims=True)

    @pl.when(i == 0)
    def _():
        sacc[...] = s
        macc[...] = m

    @pl.when(i > 0)
    def _():
        sacc[...] = sacc[...] + s
        macc[...] = jnp.maximum(macc[...], m)

    @pl.when(i == pl.num_programs(0) - 1)
    def _():
        t = _dot(sacc[...], ow1_ref[0:HID, :]) \
            + _dot(macc[...], ow1_ref[HID:2 * HID, :]) + ob1_ref[...]
        o_ref[...] = jnp.sum(_silu(t) * ow2t_ref[...], axis=1, keepdims=True) \
            + ob2_ref[...]


def _node_spec():
    return pl.BlockSpec((NODE_BLK, HID), lambda i: (i, 0))


def _w_spec(r, c):
    return pl.BlockSpec((r, c), lambda i: (0, 0))


def _input_mlp(x, W_in, b_in):
    return pl.pallas_call(
        _in_mlp_body,
        grid=(N_NODES // NODE_BLK,),
        in_specs=[_node_spec(), _w_spec(HID, HID), _w_spec(1, HID)],
        out_specs=_node_spec(),
        out_shape=jax.ShapeDtypeStruct((N_NODES, HID), jnp.float32),
    )(x, W_in, b_in.reshape(1, HID))


def _precompute(h, w1s, w1d, b1):
    return pl.pallas_call(
        _precompute_body,
        grid=(N_NODES // NODE_BLK,),
        in_specs=[_node_spec(), _w_spec(HID, HID), _w_spec(HID, HID),
                  _w_spec(1, HID)],
        out_specs=(_node_spec(), _node_spec()),
        out_shape=(jax.ShapeDtypeStruct((N_NODES, HID), jnp.float32),
                   jax.ShapeDtypeStruct((N_NODES, HID), jnp.float32)),
    )(h, w1s, w1d, b1.reshape(1, HID))


def _edge_mlp(ga, gb, rbf_p, w1r, w2, b2):
    eblk = pl.BlockSpec((EDGE_BLK, HID), lambda i: (i, 0))
    return pl.pallas_call(
        _edge_body,
        grid=(E_PAD // EDGE_BLK,),
        in_specs=[eblk, eblk,
                  pl.BlockSpec((EDGE_BLK, RBF_PAD), lambda i: (i, 0)),
                  _w_spec(RBF_PAD, HID), _w_spec(HID, HID), _w_spec(1, HID)],
        out_specs=eblk,
        out_shape=jax.ShapeDtypeStruct((E_PAD, HID), jnp.float32),
    )(ga, gb, rbf_p, w1r, w2, b2.reshape(1, HID))


def _update(s0, s1, h, u1, c1, u2, c2):
    return pl.pallas_call(
        _update_body,
        grid=(N_NODES // NODE_BLK,),
        in_specs=[_node_spec(), _node_spec(), _node_spec(),
                  _w_spec(HID, HID), _w_spec(1, HID),
                  _w_spec(HID, HID), _w_spec(1, HID)],
        out_specs=_node_spec(),
        out_shape=jax.ShapeDtypeStruct((N_NODES, HID), jnp.float32),
    )(s0, s1, h, u1, c1.reshape(1, HID), u2, c2.reshape(1, HID))


def _readout(h, won_W1, won_b1, won_W2, won_b2, out_W1, out_b1, out_W2, out_b2):
    return pl.pallas_call(
        _readout_body,
        grid=(N_NODES // NODE_BLK,),
        in_specs=[_node_spec(), _w_spec(HID, HID), _w_spec(1, HID),
                  _w_spec(HID, HID), _w_spec(1, HID),
                  _w_spec(2 * HID, HID), _w_spec(1, HID),
                  _w_spec(1, HID), _w_spec(1, 1)],
        out_specs=pl.BlockSpec((1, 1), lambda i: (0, 0)),
        out_shape=jax.ShapeDtypeStruct((1, 1), jnp.float32),
        scratch_shapes=[pltpu.VMEM((1, HID), jnp.float32),
                        pltpu.VMEM((1, HID), jnp.float32)],
    )(h, won_W1, won_b1.reshape(1, HID), won_W2, won_b2.reshape(1, HID),
      out_W1, out_b1.reshape(1, HID), out_W2.T.reshape(1, HID),
      out_b2.reshape(1, 1))


SC_NC = 2
SC_NS = 16
NW = SC_NC * SC_NS
EPW = E_PAD // NW
CH = 128
NCH = EPW // CH
ROWS_PER_TILE = N_PAD // SC_NS

GNCH0 = 112
GNCH1 = 48
GNCHMAX = max(GNCH0, GNCH1)

_SC_MESH = plsc.VectorSubcoreMesh(core_axis_name="c", subcore_axis_name="s")


@functools.partial(
    pl.kernel,
    out_type=(jax.ShapeDtypeStruct((E_PAD, HID), jnp.float32),
              jax.ShapeDtypeStruct((E_PAD, HID), jnp.float32)),
    mesh=_SC_MESH,
    scratch_types=[
        pltpu.VMEM((GNCHMAX * CH,), jnp.int32),
        pltpu.VMEM((GNCHMAX * CH,), jnp.int32),
        pltpu.VMEM((CH, HID), jnp.float32),
        pltpu.VMEM((CH, HID), jnp.float32),
        pltpu.VMEM((CH, HID), jnp.float32),
        pltpu.VMEM((CH, HID), jnp.float32),
        pltpu.SemaphoreType.DMA,
        pltpu.SemaphoreType.DMA,
        pltpu.SemaphoreType.DMA,
        pltpu.SemaphoreType.DMA,
    ],
)
def _sc_gather(a_hbm, b_hbm, src_hbm, dst_hbm, ga_hbm, gb_hbm,
               sidx_v, didx_v, ra0_v, rb0_v, ra1_v, rb1_v,
               sa0, sb0, sa1, sb1):
    cid = lax.axis_index("c")
    sid = lax.axis_index("s")
    nch = jnp.where(cid == 0, GNCH0, GNCH1)
    base = jnp.where(cid == 0, sid * (GNCH0 * CH),
                     SC_NS * (GNCH0 * CH) + sid * (GNCH1 * CH))
    pltpu.sync_copy(src_hbm.at[pl.ds(base, GNCHMAX * CH)], sidx_v)
    pltpu.sync_copy(dst_hbm.at[pl.ds(base, GNCHMAX * CH)], didx_v)

    bufs = ((ra0_v, rb0_v, sa0, sb0), (ra1_v, rb1_v, sa1, sb1))

    def issue(chunk, ra, rb, sa, sb):
        off = chunk * CH
        pltpu.async_copy(a_hbm.at[sidx_v.at[pl.ds(off, CH)]], ra, sa)
        pltpu.async_copy(b_hbm.at[didx_v.at[pl.ds(off, CH)]], rb, sb)

    for par in (0, 1):
        issue(par, *bufs[par])

    def body(jj, carry):
        for par in (0, 1):
            ra, rb, sa, sb = bufs[par]
            i = jj * 2 + par
            off = i * CH
            pltpu.make_async_copy(
                a_hbm.at[sidx_v.at[pl.ds(off, CH)]], ra, sa).wait()
            pltpu.make_async_copy(
                b_hbm.at[didx_v.at[pl.ds(off, CH)]], rb, sb).wait()
            pltpu.sync_copy(ra, ga_hbm.at[pl.ds(base + off, CH)])
            pltpu.sync_copy(rb, gb_hbm.at[pl.ds(base + off, CH)])

            @pl.when(i + 2 < nch)
            def _():
                issue(i + 2, ra, rb, sa, sb)
        return carry

    lax.fori_loop(0, nch // 2, body, 0)


@functools.partial(
    pl.kernel,
    out_type=jax.ShapeDtypeStruct((SC_NC, N_PAD, HID), jnp.float32),
    mesh=_SC_MESH,
    scratch_types=[
        pltpu.VMEM((NCH, CH), jnp.int32),
        pltpu.VMEM((CH, HID), jnp.float32),
        pltpu.VMEM((CH, HID), jnp.float32),
        pltpu.VMEM_SHARED((N_PAD, HID), jnp.float32),
        pltpu.SemaphoreType.DMA,
        pltpu.SemaphoreType.DMA,
    ],
)
def _sc_scatter(m_hbm, dst3_hbm, zero_hbm, out_hbm,
                idx_v, r0_v, r1_v, acc_sh, sl0, sl1):
    cid = lax.axis_index("c")
    sid = lax.axis_index("s")
    wid = sid * SC_NC + cid
    base = wid * EPW
    pltpu.sync_copy(dst3_hbm.at[wid], idx_v)
    pltpu.async_copy(m_hbm.at[pl.ds(base, CH)], r0_v, sl0)
    pltpu.async_copy(m_hbm.at[pl.ds(base + CH, CH)], r1_v, sl1)
    pltpu.sync_copy(zero_hbm, acc_sh.at[pl.ds(sid * ROWS_PER_TILE,
                                              ROWS_PER_TILE)])
    plsc.subcore_barrier()

    bufs = ((r0_v, sl0), (r1_v, sl1))

    def body(jj, carry):
        for par in (0, 1):
            rv, sl = bufs[par]
            i = jj * 2 + par
            off = base + i * CH
            pltpu.make_async_copy(m_hbm.at[pl.ds(off, CH)], rv, sl).wait()
            pltpu.sync_copy(rv, acc_sh.at[idx_v.at[i]], add=True)

            @pl.when(i + 2 < NCH)
            def _():
                pltpu.async_copy(m_hbm.at[pl.ds(off + 2 * CH, CH)], rv, sl)
        return carry

    lax.fori_loop(0, NCH // 2, body, 0)
    plsc.subcore_barrier()
    r0 = sid * ROWS_PER_TILE
    pltpu.sync_copy(acc_sh.at[pl.ds(r0, ROWS_PER_TILE)],
                    out_hbm.at[cid, pl.ds(r0, ROWS_PER_TILE)])


def _gather_ab(a_tab, b_tab, src_p, dst_g):
    return _sc_gather(a_tab, b_tab, src_p, dst_g)


def _scatter_sum(m, dst_p, zero_rows):
    s = _sc_scatter(m, dst_p.reshape(NW, NCH, CH), zero_rows)
    return s[0, :N_NODES], s[1, :N_NODES]


def kernel(x, edge_index, d_rbf, W_in, b_in, msg_W1, msg_b1, msg_W2, msg_b2,
           upd_W1, upd_b1, upd_W2, upd_b2, won_W1, won_b1, won_W2, won_b2,
           out_W1, out_b1, out_W2, out_b2):
    E = edge_index.shape[1]
    pad = E_PAD - E
    src_p = jnp.concatenate([edge_index[0], jnp.zeros((pad,), jnp.int32)])
    dst_p = jnp.concatenate([edge_index[1],
                             jnp.full((pad,), N_NODES, jnp.int32)])
    dst_g = jnp.concatenate([edge_index[1], jnp.zeros((pad,), jnp.int32)])
    rbf_p = jnp.zeros((E_PAD, RBF_PAD), jnp.float32).at[:E, :RBF].set(d_rbf)
    zero_rows = jnp.zeros((ROWS_PER_TILE, HID), jnp.float32)

    h = _input_mlp(x, W_in, b_in)
    n_layers = msg_W1.shape[0]
    for l in range(n_layers):
        w1s = msg_W1[l, :HID]
        w1d = msg_W1[l, HID:2 * HID]
        w1r = jnp.zeros((RBF_PAD, HID), jnp.float32).at[:RBF].set(
            msg_W1[l, 2 * HID:])
        a_tab, b_tab = _precompute(h, w1s, w1d, msg_b1[l])
        ga, gb = _gather_ab(a_tab, b_tab, src_p, dst_g)
        m = _edge_mlp(ga, gb, rbf_p, w1r, msg_W2[l], msg_b2[l])
        s0, s1 = _scatter_sum(m, dst_p, zero_rows)
        h = _update(s0, s1, h, upd_W1[l], upd_b1[l], upd_W2[l], upd_b2[l])

    return _readout(h, won_W1, won_b1, won_W2, won_b2,
                    out_W1, out_b1, out_W2, out_b2)


# speedup vs baseline: 1.0311x; 1.0090x over previous
import functools

import jax
import jax.numpy as jnp
from jax import lax
from jax.experimental import pallas as pl
from jax.experimental.pallas import tpu as pltpu
from jax.experimental.pallas import tpu_sc as plsc

N_NODES = 10000
N_PAD = 10240
HID = 128
RBF = 6
RBF_PAD = 8
NODE_BLK = 1000
EDGE_BLK = 1024
E_PAD = 327680


def _silu(x):
    return x * jax.nn.sigmoid(x)


def _dot(a, b):
    return jnp.dot(a, b, preferred_element_type=jnp.float32)


def _in_mlp_body(x_ref, w_ref, b_ref, o_ref):
    o_ref[...] = _silu(_dot(x_ref[...], w_ref[...]) + b_ref[...])


def _precompute_body(h_ref, ws_ref, wd_ref, b1_ref, a_ref, b_ref):
    h = h_ref[...]
    a_ref[...] = _dot(h, ws_ref[...]) + b1_ref[...]
    b_ref[...] = _dot(h, wd_ref[...])


def _edge_body(ga_ref, gb_ref, rbf_ref, wr_ref, w2_ref, b2_ref, m_ref):
    g = ga_ref[...] + gb_ref[...] + _dot(rbf_ref[...], wr_ref[...])
    m = _silu(g)
    m = _silu(_dot(m.astype(jnp.bfloat16), w2_ref[...]) + b2_ref[...])
    m_ref[...] = m * jax.nn.sigmoid(m)


def _update_body(s0_ref, s1_ref, h_ref, u1_ref, c1_ref, u2_ref, c2_ref, o_ref):
    h = h_ref[...]
    u = s0_ref[...] + s1_ref[...] + h
    o_ref[...] = _dot(_silu(_dot(u, u1_ref[...]) + c1_ref[...]), u2_ref[...]) \
        + c2_ref[...] + h


def _readout_body(h_ref, w1_ref, b1_ref, w2_ref, b2_ref,
                  ow1_ref, ob1_ref, ow2t_ref, ob2_ref, o_ref, sacc, macc):
    i = pl.program_id(0)
    y = _dot(_silu(_dot(h_ref[...], w1_ref[...]) + b1_ref[...]), w2_ref[...]) \
        + b2_ref[...]
    s = jnp.sum(y, axis=0, keepdims=True)
    m = jnp.max(y, axis=0, keepdims=True)

    @pl.when(i == 0)
    def _():
        sacc[...] = s
        macc[...] = m

    @pl.when(i > 0)
    def _():
        sacc[...] = sacc[...] + s
        macc[...] = jnp.maximum(macc[...], m)

    @pl.when(i == pl.num_programs(0) - 1)
    def _():
        t = _dot(sacc[...], ow1_ref[0:HID, :]) \
            + _dot(macc[...], ow1_ref[HID:2 * HID, :]) + ob1_ref[...]
        o_ref[...] = jnp.sum(_silu(t) * ow2t_ref[...], axis=1, keepdims=True) \
            + ob2_ref[...]


def _node_spec():
    return pl.BlockSpec((NODE_BLK, HID), lambda i: (i, 0))


def _w_spec(r, c):
    return pl.BlockSpec((r, c), lambda i: (0, 0))


def _input_mlp(x, W_in, b_in):
    return pl.pallas_call(
        _in_mlp_body,
        grid=(N_NODES // NODE_BLK,),
        in_specs=[_node_spec(), _w_spec(HID, HID), _w_spec(1, HID)],
        out_specs=_node_spec(),
        out_shape=jax.ShapeDtypeStruct((N_NODES, HID), jnp.float32),
    )(x, W_in, b_in.reshape(1, HID))


def _precompute(h, w1s, w1d, b1):
    return pl.pallas_call(
        _precompute_body,
        grid=(N_NODES // NODE_BLK,),
        in_specs=[_node_spec(), _w_spec(HID, HID), _w_spec(HID, HID),
                  _w_spec(1, HID)],
        out_specs=(_node_spec(), _node_spec()),
        out_shape=(jax.ShapeDtypeStruct((N_NODES, HID), jnp.float32),
                   jax.ShapeDtypeStruct((N_NODES, HID), jnp.float32)),
    )(h, w1s, w1d, b1.reshape(1, HID))


def _edge_mlp(ga, gb, rbf_p, w1r, w2, b2):
    eblk = pl.BlockSpec((EDGE_BLK, HID), lambda i: (i, 0))
    return pl.pallas_call(
        _edge_body,
        grid=(E_PAD // EDGE_BLK,),
        in_specs=[eblk, eblk,
                  pl.BlockSpec((EDGE_BLK, RBF_PAD), lambda i: (i, 0)),
                  _w_spec(RBF_PAD, HID), _w_spec(HID, HID), _w_spec(1, HID)],
        out_specs=eblk,
        out_shape=jax.ShapeDtypeStruct((E_PAD, HID), jnp.float32),
    )(ga, gb, rbf_p, w1r, w2.astype(jnp.bfloat16), b2.reshape(1, HID))


def _update(s0, s1, h, u1, c1, u2, c2):
    return pl.pallas_call(
        _update_body,
        grid=(N_NODES // NODE_BLK,),
        in_specs=[_node_spec(), _node_spec(), _node_spec(),
                  _w_spec(HID, HID), _w_spec(1, HID),
                  _w_spec(HID, HID), _w_spec(1, HID)],
        out_specs=_node_spec(),
        out_shape=jax.ShapeDtypeStruct((N_NODES, HID), jnp.float32),
    )(s0, s1, h, u1, c1.reshape(1, HID), u2, c2.reshape(1, HID))


def _readout(h, won_W1, won_b1, won_W2, won_b2, out_W1, out_b1, out_W2, out_b2):
    return pl.pallas_call(
        _readout_body,
        grid=(N_NODES // NODE_BLK,),
        in_specs=[_node_spec(), _w_spec(HID, HID), _w_spec(1, HID),
                  _w_spec(HID, HID), _w_spec(1, HID),
                  _w_spec(2 * HID, HID), _w_spec(1, HID),
                  _w_spec(1, HID), _w_spec(1, 1)],
        out_specs=pl.BlockSpec((1, 1), lambda i: (0, 0)),
        out_shape=jax.ShapeDtypeStruct((1, 1), jnp.float32),
        scratch_shapes=[pltpu.VMEM((1, HID), jnp.float32),
                        pltpu.VMEM((1, HID), jnp.float32)],
    )(h, won_W1, won_b1.reshape(1, HID), won_W2, won_b2.reshape(1, HID),
      out_W1, out_b1.reshape(1, HID), out_W2.T.reshape(1, HID),
      out_b2.reshape(1, 1))


SC_NC = 2
SC_NS = 16
NW = SC_NC * SC_NS
EPW = E_PAD // NW
CH = 128
NCH = EPW // CH
ROWS_PER_TILE = N_PAD // SC_NS

GNCH0 = 80
GNCH1 = 80
GNCHMAX = max(GNCH0, GNCH1)

_SC_MESH = plsc.VectorSubcoreMesh(core_axis_name="c", subcore_axis_name="s")


@functools.partial(
    pl.kernel,
    out_type=(jax.ShapeDtypeStruct((E_PAD, HID), jnp.float32),
              jax.ShapeDtypeStruct((E_PAD, HID), jnp.float32)),
    mesh=_SC_MESH,
    scratch_types=[
        pltpu.VMEM((GNCHMAX * CH,), jnp.int32),
        pltpu.VMEM((GNCHMAX * CH,), jnp.int32),
        pltpu.VMEM((CH, HID), jnp.float32),
        pltpu.VMEM((CH, HID), jnp.float32),
        pltpu.VMEM((CH, HID), jnp.float32),
        pltpu.VMEM((CH, HID), jnp.float32),
        pltpu.SemaphoreType.DMA,
        pltpu.SemaphoreType.DMA,
        pltpu.SemaphoreType.DMA,
        pltpu.SemaphoreType.DMA,
    ],
)
def _sc_gather(a_hbm, b_hbm, src_hbm, dst_hbm, ga_hbm, gb_hbm,
               sidx_v, didx_v, ra0_v, rb0_v, ra1_v, rb1_v,
               sa0, sb0, sa1, sb1):
    cid = lax.axis_index("c")
    sid = lax.axis_index("s")
    nch = jnp.where(cid == 0, GNCH0, GNCH1)
    base = jnp.where(cid == 0, sid * (GNCH0 * CH),
                     SC_NS * (GNCH0 * CH) + sid * (GNCH1 * CH))
    pltpu.sync_copy(src_hbm.at[pl.ds(base, GNCHMAX * CH)], sidx_v)
    pltpu.sync_copy(dst_hbm.at[pl.ds(base, GNCHMAX * CH)], didx_v)

    bufs = ((ra0_v, rb0_v, sa0, sb0), (ra1_v, rb1_v, sa1, sb1))

    def issue(chunk, ra, rb, sa, sb):
        off = chunk * CH
        pltpu.async_copy(a_hbm.at[sidx_v.at[pl.ds(off, CH)]], ra, sa)
        pltpu.async_copy(b_hbm.at[didx_v.at[pl.ds(off, CH)]], rb, sb)

    for par in (0, 1):
        issue(par, *bufs[par])

    def body(jj, carry):
        for par in (0, 1):
            ra, rb, sa, sb = bufs[par]
            i = jj * 2 + par
            off = i * CH
            pltpu.make_async_copy(
                a_hbm.at[sidx_v.at[pl.ds(off, CH)]], ra, sa).wait()
            pltpu.make_async_copy(
                b_hbm.at[didx_v.at[pl.ds(off, CH)]], rb, sb).wait()
            pltpu.sync_copy(ra, ga_hbm.at[pl.ds(base + off, CH)])
            pltpu.sync_copy(rb, gb_hbm.at[pl.ds(base + off, CH)])

            @pl.when(i + 2 < nch)
            def _():
                issue(i + 2, ra, rb, sa, sb)
        return carry

    lax.fori_loop(0, nch // 2, body, 0)


@functools.partial(
    pl.kernel,
    out_type=jax.ShapeDtypeStruct((SC_NC, N_PAD, HID), jnp.float32),
    mesh=_SC_MESH,
    scratch_types=[
        pltpu.VMEM((NCH, CH), jnp.int32),
        pltpu.VMEM((CH, HID), jnp.float32),
        pltpu.VMEM((CH, HID), jnp.float32),
        pltpu.VMEM_SHARED((N_PAD, HID), jnp.float32),
        pltpu.SemaphoreType.DMA,
        pltpu.SemaphoreType.DMA,
    ],
)
def _sc_scatter(m_hbm, dst3_hbm, zero_hbm, out_hbm,
                idx_v, r0_v, r1_v, acc_sh, sl0, sl1):
    cid = lax.axis_index("c")
    sid = lax.axis_index("s")
    wid = sid * SC_NC + cid
    base = wid * EPW
    pltpu.sync_copy(dst3_hbm.at[wid], idx_v)
    pltpu.async_copy(m_hbm.at[pl.ds(base, CH)], r0_v, sl0)
    pltpu.async_copy(m_hbm.at[pl.ds(base + CH, CH)], r1_v, sl1)
    pltpu.sync_copy(zero_hbm, acc_sh.at[pl.ds(sid * ROWS_PER_TILE,
                                              ROWS_PER_TILE)])
    plsc.subcore_barrier()

    bufs = ((r0_v, sl0), (r1_v, sl1))

    def body(jj, carry):
        for par in (0, 1):
            rv, sl = bufs[par]
            i = jj * 2 + par
            off = base + i * CH
            pltpu.make_async_copy(m_hbm.at[pl.ds(off, CH)], rv, sl).wait()
            pltpu.sync_copy(rv, acc_sh.at[idx_v.at[i]], add=True)

            @pl.when(i + 2 < NCH)
            def _():
                pltpu.async_copy(m_hbm.at[pl.ds(off + 2 * CH, CH)], rv, sl)
        return carry

    lax.fori_loop(0, NCH // 2, body, 0)
    plsc.subcore_barrier()
    r0 = sid * ROWS_PER_TILE
    pltpu.sync_copy(acc_sh.at[pl.ds(r0, ROWS_PER_TILE)],
                    out_hbm.at[cid, pl.ds(r0, ROWS_PER_TILE)])


def _gather_ab(a_tab, b_tab, src_p, dst_g):
    return _sc_gather(a_tab, b_tab, src_p, dst_g)


def _scatter_sum(m, dst_p, zero_rows):
    s = _sc_scatter(m, dst_p.reshape(NW, NCH, CH), zero_rows)
    return s[0, :N_NODES], s[1, :N_NODES]


def kernel(x, edge_index, d_rbf, W_in, b_in, msg_W1, msg_b1, msg_W2, msg_b2,
           upd_W1, upd_b1, upd_W2, upd_b2, won_W1, won_b1, won_W2, won_b2,
           out_W1, out_b1, out_W2, out_b2):
    E = edge_index.shape[1]
    pad = E_PAD - E
    src_p = jnp.concatenate([edge_index[0], jnp.zeros((pad,), jnp.int32)])
    dst_p = jnp.concatenate([edge_index[1],
                             jnp.full((pad,), N_NODES, jnp.int32)])
    dst_g = jnp.concatenate([edge_index[1], jnp.zeros((pad,), jnp.int32)])
    rbf_p = jnp.zeros((E_PAD, RBF_PAD), jnp.float32).at[:E, :RBF].set(d_rbf)
    zero_rows = jnp.zeros((ROWS_PER_TILE, HID), jnp.float32)

    h = _input_mlp(x, W_in, b_in)
    n_layers = msg_W1.shape[0]
    for l in range(n_layers):
        w1s = msg_W1[l, :HID]
        w1d = msg_W1[l, HID:2 * HID]
        w1r = jnp.zeros((RBF_PAD, HID), jnp.float32).at[:RBF].set(
            msg_W1[l, 2 * HID:])
        a_tab, b_tab = _precompute(h, w1s, w1d, msg_b1[l])
        ga, gb = _gather_ab(a_tab, b_tab, src_p, dst_g)
        m = _edge_mlp(ga, gb, rbf_p, w1r, msg_W2[l], msg_b2[l])
        s0, s1 = _scatter_sum(m, dst_p, zero_rows)
        h = _update(s0, s1, h, upd_W1[l], upd_b1[l], upd_W2[l], upd_b2[l])

    return _readout(h, won_W1, won_b1, won_W2, won_b2,
                    out_W1, out_b1, out_W2, out_b2)


# speedup vs baseline: 1.1145x; 1.0808x over previous
import functools

import jax
import jax.numpy as jnp
from jax import lax
from jax.experimental import pallas as pl
from jax.experimental.pallas import tpu as pltpu
from jax.experimental.pallas import tpu_sc as plsc

N_NODES = 10000
N_PAD = 10240
HID = 128
RBF = 6
RBF_PAD = 8
NODE_BLK = 1000
EDGE_BLK = 1024
E_PAD = 327680


def _silu(x):
    return x * jax.nn.sigmoid(x)


def _dot(a, b):
    return jnp.dot(a, b, preferred_element_type=jnp.float32)


def _in_mlp_body(x_ref, w_ref, b_ref, o_ref):
    o_ref[...] = _silu(_dot(x_ref[...], w_ref[...]) + b_ref[...])


def _precompute_body(h_ref, ws_ref, wd_ref, b1_ref, a_ref, b_ref):
    h = h_ref[...]
    a_ref[...] = _dot(h, ws_ref[...]) + b1_ref[...]
    b_ref[...] = _dot(h, wd_ref[...])


def _edge_body(g_ref, rbf_ref, wr_ref, w2_ref, b2_ref, m_ref):
    g = g_ref[...] + _dot(rbf_ref[...], wr_ref[...])
    m = _silu(g)
    m = _silu(_dot(m.astype(jnp.bfloat16), w2_ref[...]) + b2_ref[...])
    m_ref[...] = m * jax.nn.sigmoid(m)


def _update_body(s0_ref, s1_ref, h_ref, u1_ref, c1_ref, u2_ref, c2_ref, o_ref):
    h = h_ref[...]
    u = s0_ref[...] + s1_ref[...] + h
    o_ref[...] = _dot(_silu(_dot(u, u1_ref[...]) + c1_ref[...]), u2_ref[...]) \
        + c2_ref[...] + h


def _readout_body(h_ref, w1_ref, b1_ref, w2_ref, b2_ref,
                  ow1_ref, ob1_ref, ow2t_ref, ob2_ref, o_ref, sacc, macc):
    i = pl.program_id(0)
    y = _dot(_silu(_dot(h_ref[...], w1_ref[...]) + b1_ref[...]), w2_ref[...]) \
        + b2_ref[...]
    s = jnp.sum(y, axis=0, keepdims=True)
    m = jnp.max(y, axis=0, keepdims=True)

    @pl.when(i == 0)
    def _():
        sacc[...] = s
        macc[...] = m

    @pl.when(i > 0)
    def _():
        sacc[...] = sacc[...] + s
        macc[...] = jnp.maximum(macc[...], m)

    @pl.when(i == pl.num_programs(0) - 1)
    def _():
        t = _dot(sacc[...], ow1_ref[0:HID, :]) \
            + _dot(macc[...], ow1_ref[HID:2 * HID, :]) + ob1_ref[...]
        o_ref[...] = jnp.sum(_silu(t) * ow2t_ref[...], axis=1, keepdims=True) \
            + ob2_ref[...]


def _node_spec():
    return pl.BlockSpec((NODE_BLK, HID), lambda i: (i, 0))


def _w_spec(r, c):
    return pl.BlockSpec((r, c), lambda i: (0, 0))


def _input_mlp(x, W_in, b_in):
    return pl.pallas_call(
        _in_mlp_body,
        grid=(N_NODES // NODE_BLK,),
        in_specs=[_node_spec(), _w_spec(HID, HID), _w_spec(1, HID)],
        out_specs=_node_spec(),
        out_shape=jax.ShapeDtypeStruct((N_NODES, HID), jnp.float32),
    )(x, W_in, b_in.reshape(1, HID))


def _precompute(h, w1s, w1d, b1):
    return pl.pallas_call(
        _precompute_body,
        grid=(N_NODES // NODE_BLK,),
        in_specs=[_node_spec(), _w_spec(HID, HID), _w_spec(HID, HID),
                  _w_spec(1, HID)],
        out_specs=(_node_spec(), _node_spec()),
        out_shape=(jax.ShapeDtypeStruct((N_NODES, HID), jnp.float32),
                   jax.ShapeDtypeStruct((N_NODES, HID), jnp.float32)),
    )(h, w1s, w1d, b1.reshape(1, HID))


def _edge_mlp(g, rbf_p, w1r, w2, b2):
    eblk = pl.BlockSpec((EDGE_BLK, HID), lambda i: (i, 0))
    return pl.pallas_call(
        _edge_body,
        grid=(E_PAD // EDGE_BLK,),
        in_specs=[eblk,
                  pl.BlockSpec((EDGE_BLK, RBF_PAD), lambda i: (i, 0)),
                  _w_spec(RBF_PAD, HID), _w_spec(HID, HID), _w_spec(1, HID)],
        out_specs=eblk,
        out_shape=jax.ShapeDtypeStruct((E_PAD, HID), jnp.float32),
    )(g, rbf_p, w1r, w2.astype(jnp.bfloat16), b2.reshape(1, HID))


def _update(s0, s1, h, u1, c1, u2, c2):
    return pl.pallas_call(
        _update_body,
        grid=(N_NODES // NODE_BLK,),
        in_specs=[_node_spec(), _node_spec(), _node_spec(),
                  _w_spec(HID, HID), _w_spec(1, HID),
                  _w_spec(HID, HID), _w_spec(1, HID)],
        out_specs=_node_spec(),
        out_shape=jax.ShapeDtypeStruct((N_NODES, HID), jnp.float32),
    )(s0, s1, h, u1, c1.reshape(1, HID), u2, c2.reshape(1, HID))


def _readout(h, won_W1, won_b1, won_W2, won_b2, out_W1, out_b1, out_W2, out_b2):
    return pl.pallas_call(
        _readout_body,
        grid=(N_NODES // NODE_BLK,),
        in_specs=[_node_spec(), _w_spec(HID, HID), _w_spec(1, HID),
                  _w_spec(HID, HID), _w_spec(1, HID),
                  _w_spec(2 * HID, HID), _w_spec(1, HID),
                  _w_spec(1, HID), _w_spec(1, 1)],
        out_specs=pl.BlockSpec((1, 1), lambda i: (0, 0)),
        out_shape=jax.ShapeDtypeStruct((1, 1), jnp.float32),
        scratch_shapes=[pltpu.VMEM((1, HID), jnp.float32),
                        pltpu.VMEM((1, HID), jnp.float32)],
    )(h, won_W1, won_b1.reshape(1, HID), won_W2, won_b2.reshape(1, HID),
      out_W1, out_b1.reshape(1, HID), out_W2.T.reshape(1, HID),
      out_b2.reshape(1, 1))


SC_NC = 2
SC_NS = 16
NW = SC_NC * SC_NS
EPW = E_PAD // NW
CH = 128
NCH = EPW // CH
ROWS_PER_TILE = N_PAD // SC_NS

GNCH0 = 80
GNCH1 = 80
GNCHMAX = max(GNCH0, GNCH1)

_SC_MESH = plsc.VectorSubcoreMesh(core_axis_name="c", subcore_axis_name="s")


@functools.partial(
    pl.kernel,
    out_type=jax.ShapeDtypeStruct((E_PAD, HID), jnp.float32),
    mesh=_SC_MESH,
    scratch_types=[
        pltpu.VMEM((GNCHMAX * CH,), jnp.int32),
        pltpu.VMEM((GNCHMAX * CH,), jnp.int32),
        pltpu.VMEM((CH, HID), jnp.float32),
        pltpu.VMEM((CH, HID), jnp.float32),
        pltpu.VMEM((CH, HID), jnp.float32),
        pltpu.VMEM((CH, HID), jnp.float32),
        pltpu.SemaphoreType.DMA,
        pltpu.SemaphoreType.DMA,
        pltpu.SemaphoreType.DMA,
        pltpu.SemaphoreType.DMA,
    ],
)
def _sc_gather(a_hbm, b_hbm, src_hbm, dst_hbm, g_hbm,
               sidx_v, didx_v, ra0_v, rb0_v, ra1_v, rb1_v,
               sa0, sb0, sa1, sb1):
    cid = lax.axis_index("c")
    sid = lax.axis_index("s")
    nch = jnp.where(cid == 0, GNCH0, GNCH1)
    base = jnp.where(cid == 0, sid * (GNCH0 * CH),
                     SC_NS * (GNCH0 * CH) + sid * (GNCH1 * CH))
    pltpu.sync_copy(src_hbm.at[pl.ds(base, GNCHMAX * CH)], sidx_v)
    pltpu.sync_copy(dst_hbm.at[pl.ds(base, GNCHMAX * CH)], didx_v)

    bufs = ((ra0_v, rb0_v, sa0, sb0), (ra1_v, rb1_v, sa1, sb1))

    def issue(chunk, ra, rb, sa, sb):
        off = chunk * CH
        pltpu.async_copy(a_hbm.at[sidx_v.at[pl.ds(off, CH)]], ra, sa)
        pltpu.async_copy(b_hbm.at[didx_v.at[pl.ds(off, CH)]], rb, sb)

    for par in (0, 1):
        issue(par, *bufs[par])

    def body(jj, carry):
        for par in (0, 1):
            ra, rb, sa, sb = bufs[par]
            i = jj * 2 + par
            off = i * CH
            pltpu.make_async_copy(
                a_hbm.at[sidx_v.at[pl.ds(off, CH)]], ra, sa).wait()
            pltpu.make_async_copy(
                b_hbm.at[didx_v.at[pl.ds(off, CH)]], rb, sb).wait()

            def add_row(r, c2):
                for c in range(HID // 16):
                    sl = pl.ds(c * 16, 16)
                    ra[r, sl] = ra[r, sl] + rb[r, sl]
                return c2

            lax.fori_loop(0, CH, add_row, 0)
            pltpu.sync_copy(ra, g_hbm.at[pl.ds(base + off, CH)])

            @pl.when(i + 2 < nch)
            def _():
                issue(i + 2, ra, rb, sa, sb)
        return carry

    lax.fori_loop(0, nch // 2, body, 0)


@functools.partial(
    pl.kernel,
    out_type=jax.ShapeDtypeStruct((SC_NC, N_PAD, HID), jnp.float32),
    mesh=_SC_MESH,
    scratch_types=[
        pltpu.VMEM((NCH, CH), jnp.int32),
        pltpu.VMEM((CH, HID), jnp.float32),
        pltpu.VMEM((CH, HID), jnp.float32),
        pltpu.VMEM_SHARED((N_PAD, HID), jnp.float32),
        pltpu.SemaphoreType.DMA,
        pltpu.SemaphoreType.DMA,
    ],
)
def _sc_scatter(m_hbm, dst3_hbm, zero_hbm, out_hbm,
                idx_v, r0_v, r1_v, acc_sh, sl0, sl1):
    cid = lax.axis_index("c")
    sid = lax.axis_index("s")
    wid = sid * SC_NC + cid
    base = wid * EPW
    pltpu.sync_copy(dst3_hbm.at[wid], idx_v)
    pltpu.async_copy(m_hbm.at[pl.ds(base, CH)], r0_v, sl0)
    pltpu.async_copy(m_hbm.at[pl.ds(base + CH, CH)], r1_v, sl1)
    pltpu.sync_copy(zero_hbm, acc_sh.at[pl.ds(sid * ROWS_PER_TILE,
                                              ROWS_PER_TILE)])
    plsc.subcore_barrier()

    bufs = ((r0_v, sl0), (r1_v, sl1))

    def body(jj, carry):
        for par in (0, 1):
            rv, sl = bufs[par]
            i = jj * 2 + par
            off = base + i * CH
            pltpu.make_async_copy(m_hbm.at[pl.ds(off, CH)], rv, sl).wait()
            pltpu.sync_copy(rv, acc_sh.at[idx_v.at[i]], add=True)

            @pl.when(i + 2 < NCH)
            def _():
                pltpu.async_copy(m_hbm.at[pl.ds(off + 2 * CH, CH)], rv, sl)
        return carry

    lax.fori_loop(0, NCH // 2, body, 0)
    plsc.subcore_barrier()
    r0 = sid * ROWS_PER_TILE
    pltpu.sync_copy(acc_sh.at[pl.ds(r0, ROWS_PER_TILE)],
                    out_hbm.at[cid, pl.ds(r0, ROWS_PER_TILE)])


def _gather_ab(a_tab, b_tab, src_p, dst_g):
    return _sc_gather(a_tab, b_tab, src_p, dst_g)


def _scatter_sum(m, dst_p, zero_rows):
    s = _sc_scatter(m, dst_p.reshape(NW, NCH, CH), zero_rows)
    return s[0, :N_NODES], s[1, :N_NODES]


def kernel(x, edge_index, d_rbf, W_in, b_in, msg_W1, msg_b1, msg_W2, msg_b2,
           upd_W1, upd_b1, upd_W2, upd_b2, won_W1, won_b1, won_W2, won_b2,
           out_W1, out_b1, out_W2, out_b2):
    E = edge_index.shape[1]
    pad = E_PAD - E
    src_p = jnp.concatenate([edge_index[0], jnp.zeros((pad,), jnp.int32)])
    dst_p = jnp.concatenate([edge_index[1],
                             jnp.full((pad,), N_NODES, jnp.int32)])
    dst_g = jnp.concatenate([edge_index[1], jnp.zeros((pad,), jnp.int32)])
    rbf_p = jnp.zeros((E_PAD, RBF_PAD), jnp.float32).at[:E, :RBF].set(d_rbf)
    zero_rows = jnp.zeros((ROWS_PER_TILE, HID), jnp.float32)

    h = _input_mlp(x, W_in, b_in)
    n_layers = msg_W1.shape[0]
    for l in range(n_layers):
        w1s = msg_W1[l, :HID]
        w1d = msg_W1[l, HID:2 * HID]
        w1r = jnp.zeros((RBF_PAD, HID), jnp.float32).at[:RBF].set(
            msg_W1[l, 2 * HID:])
        a_tab, b_tab = _precompute(h, w1s, w1d, msg_b1[l])
        g = _gather_ab(a_tab, b_tab, src_p, dst_g)
        m = _edge_mlp(g, rbf_p, w1r, msg_W2[l], msg_b2[l])
        s0, s1 = _scatter_sum(m, dst_p, zero_rows)
        h = _update(s0, s1, h, upd_W1[l], upd_b1[l], upd_W2[l], upd_b2[l])

    return _readout(h, won_W1, won_b1, won_W2, won_b2,
                    out_W1, out_b1, out_W2, out_b2)


# speedup vs baseline: 1.3042x; 1.1703x over previous
import functools

import jax
import jax.numpy as jnp
from jax import lax
from jax.experimental import pallas as pl
from jax.experimental.pallas import tpu as pltpu
from jax.experimental.pallas import tpu_sc as plsc

N_NODES = 10000
N_PAD = 10240
HID = 128
RBF = 6
RBF_PAD = 8
NODE_BLK = 1000
EDGE_BLK = 1024
E_PAD = 327680


def _silu(x):
    return x * jax.nn.sigmoid(x)


def _dot(a, b):
    return jnp.dot(a, b, preferred_element_type=jnp.float32)


def _in_mlp_body(x_ref, w_ref, b_ref, o_ref):
    o_ref[...] = _silu(_dot(x_ref[...], w_ref[...]) + b_ref[...])


def _precompute_body(h_ref, ws_ref, wd_ref, b1_ref, a_ref, b_ref):
    h = h_ref[...]
    a_ref[...] = _dot(h, ws_ref[...]) + b1_ref[...]
    b_ref[...] = _dot(h, wd_ref[...])


def _edge_body(g_ref, rbf_ref, wr_ref, w2_ref, b2_ref, m_ref):
    g = g_ref[...] + _dot(rbf_ref[...], wr_ref[...])
    m = _silu(g)
    m = _silu(_dot(m.astype(jnp.bfloat16), w2_ref[...]) + b2_ref[...])
    m_ref[...] = m * jax.nn.sigmoid(m)


def _update_body(s0_ref, s1_ref, s2_ref, s3_ref, h_ref,
                 u1_ref, c1_ref, u2_ref, c2_ref, o_ref):
    h = h_ref[...]
    u = s0_ref[...] + s1_ref[...] + s2_ref[...] + s3_ref[...] + h
    o_ref[...] = _dot(_silu(_dot(u, u1_ref[...]) + c1_ref[...]), u2_ref[...]) \
        + c2_ref[...] + h


def _readout_body(h_ref, w1_ref, b1_ref, w2_ref, b2_ref,
                  ow1_ref, ob1_ref, ow2t_ref, ob2_ref, o_ref, sacc, macc):
    i = pl.program_id(0)
    y = _dot(_silu(_dot(h_ref[...], w1_ref[...]) + b1_ref[...]), w2_ref[...]) \
        + b2_ref[...]
    s = jnp.sum(y, axis=0, keepdims=True)
    m = jnp.max(y, axis=0, keepdims=True)

    @pl.when(i == 0)
    def _():
        sacc[...] = s
        macc[...] = m

    @pl.when(i > 0)
    def _():
        sacc[...] = sacc[...] + s
        macc[...] = jnp.maximum(macc[...], m)

    @pl.when(i == pl.num_programs(0) - 1)
    def _():
        t = _dot(sacc[...], ow1_ref[0:HID, :]) \
            + _dot(macc[...], ow1_ref[HID:2 * HID, :]) + ob1_ref[...]
        o_ref[...] = jnp.sum(_silu(t) * ow2t_ref[...], axis=1, keepdims=True) \
            + ob2_ref[...]


def _node_spec():
    return pl.BlockSpec((NODE_BLK, HID), lambda i: (i, 0))


def _w_spec(r, c):
    return pl.BlockSpec((r, c), lambda i: (0, 0))


def _input_mlp(x, W_in, b_in):
    return pl.pallas_call(
        _in_mlp_body,
        grid=(N_NODES // NODE_BLK,),
        in_specs=[_node_spec(), _w_spec(HID, HID), _w_spec(1, HID)],
        out_specs=_node_spec(),
        out_shape=jax.ShapeDtypeStruct((N_NODES, HID), jnp.float32),
    )(x, W_in, b_in.reshape(1, HID))


def _precompute(h, w1s, w1d, b1):
    return pl.pallas_call(
        _precompute_body,
        grid=(N_NODES // NODE_BLK,),
        in_specs=[_node_spec(), _w_spec(HID, HID), _w_spec(HID, HID),
                  _w_spec(1, HID)],
        out_specs=(_node_spec(), _node_spec()),
        out_shape=(jax.ShapeDtypeStruct((N_NODES, HID), jnp.float32),
                   jax.ShapeDtypeStruct((N_NODES, HID), jnp.float32)),
    )(h, w1s, w1d, b1.reshape(1, HID))


def _edge_mlp(g, rbf_p, w1r, w2, b2):
    eblk = pl.BlockSpec((EDGE_BLK, HID), lambda i: (i, 0))
    return pl.pallas_call(
        _edge_body,
        grid=(E_HALF // EDGE_BLK,),
        in_specs=[eblk,
                  pl.BlockSpec((EDGE_BLK, RBF_PAD), lambda i: (i, 0)),
                  _w_spec(RBF_PAD, HID), _w_spec(HID, HID), _w_spec(1, HID)],
        out_specs=eblk,
        out_shape=jax.ShapeDtypeStruct((E_HALF, HID), jnp.float32),
    )(g, rbf_p, w1r, w2.astype(jnp.bfloat16), b2.reshape(1, HID))


def _update(s0, s1, s2, s3, h, u1, c1, u2, c2):
    return pl.pallas_call(
        _update_body,
        grid=(N_NODES // NODE_BLK,),
        in_specs=[_node_spec(), _node_spec(), _node_spec(), _node_spec(),
                  _node_spec(),
                  _w_spec(HID, HID), _w_spec(1, HID),
                  _w_spec(HID, HID), _w_spec(1, HID)],
        out_specs=_node_spec(),
        out_shape=jax.ShapeDtypeStruct((N_NODES, HID), jnp.float32),
    )(s0, s1, s2, s3, h, u1, c1.reshape(1, HID), u2, c2.reshape(1, HID))


def _readout(h, won_W1, won_b1, won_W2, won_b2, out_W1, out_b1, out_W2, out_b2):
    return pl.pallas_call(
        _readout_body,
        grid=(N_NODES // NODE_BLK,),
        in_specs=[_node_spec(), _w_spec(HID, HID), _w_spec(1, HID),
                  _w_spec(HID, HID), _w_spec(1, HID),
                  _w_spec(2 * HID, HID), _w_spec(1, HID),
                  _w_spec(1, HID), _w_spec(1, 1)],
        out_specs=pl.BlockSpec((1, 1), lambda i: (0, 0)),
        out_shape=jax.ShapeDtypeStruct((1, 1), jnp.float32),
        scratch_shapes=[pltpu.VMEM((1, HID), jnp.float32),
                        pltpu.VMEM((1, HID), jnp.float32)],
    )(h, won_W1, won_b1.reshape(1, HID), won_W2, won_b2.reshape(1, HID),
      out_W1, out_b1.reshape(1, HID), out_W2.T.reshape(1, HID),
      out_b2.reshape(1, 1))


SC_NC = 2
SC_NS = 16
NW = SC_NC * SC_NS
CH = 128
E_HALF = E_PAD // 2
NCH_H = E_HALF // (NW * CH)
ROWS_PER_TILE = N_PAD // SC_NS

_SC_MESH = plsc.VectorSubcoreMesh(core_axis_name="c", subcore_axis_name="s")


@functools.partial(
    pl.kernel,
    out_type=jax.ShapeDtypeStruct((E_HALF, HID), jnp.float32),
    mesh=_SC_MESH,
    scratch_types=[
        pltpu.VMEM((NCH_H * CH,), jnp.int32),
        pltpu.VMEM((NCH_H * CH,), jnp.int32),
        pltpu.VMEM((CH, HID), jnp.float32),
        pltpu.VMEM((CH, HID), jnp.float32),
        pltpu.VMEM((CH, HID), jnp.float32),
        pltpu.VMEM((CH, HID), jnp.float32),
        pltpu.SemaphoreType.DMA,
        pltpu.SemaphoreType.DMA,
        pltpu.SemaphoreType.DMA,
        pltpu.SemaphoreType.DMA,
    ],
)
def _sc_gather(a_hbm, b_hbm, src_hbm, dst_hbm, g_hbm,
               sidx_v, didx_v, ra0_v, rb0_v, ra1_v, rb1_v,
               sa0, sb0, sa1, sb1):
    wid = lax.axis_index("s") * SC_NC + lax.axis_index("c")
    base = wid * (NCH_H * CH)
    pltpu.sync_copy(src_hbm.at[pl.ds(base, NCH_H * CH)], sidx_v)
    pltpu.sync_copy(dst_hbm.at[pl.ds(base, NCH_H * CH)], didx_v)

    bufs = ((ra0_v, rb0_v, sa0, sb0), (ra1_v, rb1_v, sa1, sb1))

    def issue(chunk, ra, rb, sa, sb):
        off = chunk * CH
        pltpu.async_copy(a_hbm.at[sidx_v.at[pl.ds(off, CH)]], ra, sa)
        pltpu.async_copy(b_hbm.at[didx_v.at[pl.ds(off, CH)]], rb, sb)

    for par in (0, 1):
        issue(par, *bufs[par])

    def body(jj, carry):
        for par in (0, 1):
            ra, rb, sa, sb = bufs[par]
            i = jj * 2 + par
            off = i * CH
            pltpu.make_async_copy(
                a_hbm.at[sidx_v.at[pl.ds(off, CH)]], ra, sa).wait()
            pltpu.make_async_copy(
                b_hbm.at[didx_v.at[pl.ds(off, CH)]], rb, sb).wait()

            def add_row(r, c2):
                for c in range(HID // 16):
                    sl = pl.ds(c * 16, 16)
                    ra[r, sl] = ra[r, sl] + rb[r, sl]
                return c2

            lax.fori_loop(0, CH, add_row, 0)
            pltpu.sync_copy(ra, g_hbm.at[pl.ds(base + off, CH)])

            @pl.when(i + 2 < NCH_H)
            def _():
                issue(i + 2, ra, rb, sa, sb)
        return carry

    lax.fori_loop(0, NCH_H // 2, body, 0)


@functools.partial(
    pl.kernel,
    out_type=jax.ShapeDtypeStruct((SC_NC, N_PAD, HID), jnp.float32),
    mesh=_SC_MESH,
    scratch_types=[
        pltpu.VMEM((NCH_H, CH), jnp.int32),
        pltpu.VMEM((CH, HID), jnp.float32),
        pltpu.VMEM((CH, HID), jnp.float32),
        pltpu.VMEM_SHARED((N_PAD, HID), jnp.float32),
        pltpu.SemaphoreType.DMA,
        pltpu.SemaphoreType.DMA,
    ],
)
def _sc_scatter(m_hbm, dst3_hbm, zero_hbm, out_hbm,
                idx_v, r0_v, r1_v, acc_sh, sl0, sl1):
    cid = lax.axis_index("c")
    sid = lax.axis_index("s")
    wid = sid * SC_NC + cid
    base = wid * (NCH_H * CH)
    pltpu.sync_copy(dst3_hbm.at[wid], idx_v)
    pltpu.async_copy(m_hbm.at[pl.ds(base, CH)], r0_v, sl0)
    pltpu.async_copy(m_hbm.at[pl.ds(base + CH, CH)], r1_v, sl1)
    pltpu.sync_copy(zero_hbm, acc_sh.at[pl.ds(sid * ROWS_PER_TILE,
                                              ROWS_PER_TILE)])
    plsc.subcore_barrier()

    bufs = ((r0_v, sl0), (r1_v, sl1))

    def body(jj, carry):
        for par in (0, 1):
            rv, sl = bufs[par]
            i = jj * 2 + par
            off = base + i * CH
            pltpu.make_async_copy(m_hbm.at[pl.ds(off, CH)], rv, sl).wait()
            pltpu.sync_copy(rv, acc_sh.at[idx_v.at[i]], add=True)

            @pl.when(i + 2 < NCH_H)
            def _():
                pltpu.async_copy(m_hbm.at[pl.ds(off + 2 * CH, CH)], rv, sl)
        return carry

    lax.fori_loop(0, NCH_H // 2, body, 0)
    plsc.subcore_barrier()
    r0 = sid * ROWS_PER_TILE
    pltpu.sync_copy(acc_sh.at[pl.ds(r0, ROWS_PER_TILE)],
                    out_hbm.at[cid, pl.ds(r0, ROWS_PER_TILE)])


def _gather_ab(a_tab, b_tab, src_p, dst_g):
    return _sc_gather(a_tab, b_tab, src_p, dst_g)


def _scatter_sum(m, dst_h, zero_rows):
    s = _sc_scatter(m, dst_h.reshape(NW, NCH_H, CH), zero_rows)
    return s[0, :N_NODES], s[1, :N_NODES]


def kernel(x, edge_index, d_rbf, W_in, b_in, msg_W1, msg_b1, msg_W2, msg_b2,
           upd_W1, upd_b1, upd_W2, upd_b2, won_W1, won_b1, won_W2, won_b2,
           out_W1, out_b1, out_W2, out_b2):
    E = edge_index.shape[1]
    pad = E_PAD - E
    src_p = jnp.concatenate([edge_index[0], jnp.zeros((pad,), jnp.int32)])
    dst_p = jnp.concatenate([edge_index[1],
                             jnp.full((pad,), N_NODES, jnp.int32)])
    dst_g = jnp.concatenate([edge_index[1], jnp.zeros((pad,), jnp.int32)])
    rbf_p = jnp.zeros((E_PAD, RBF_PAD), jnp.float32).at[:E, :RBF].set(d_rbf)
    zero_rows = jnp.zeros((ROWS_PER_TILE, HID), jnp.float32)

    src_h = (src_p[:E_HALF], src_p[E_HALF:])
    dst_gh = (dst_g[:E_HALF], dst_g[E_HALF:])
    dst_ph = (dst_p[:E_HALF], dst_p[E_HALF:])
    rbf_h = (rbf_p[:E_HALF], rbf_p[E_HALF:])

    h = _input_mlp(x, W_in, b_in)
    n_layers = msg_W1.shape[0]
    for l in range(n_layers):
        w1s = msg_W1[l, :HID]
        w1d = msg_W1[l, HID:2 * HID]
        w1r = jnp.zeros((RBF_PAD, HID), jnp.float32).at[:RBF].set(
            msg_W1[l, 2 * HID:])
        a_tab, b_tab = _precompute(h, w1s, w1d, msg_b1[l])
        parts = []
        ms = []
        for half in range(2):
            g = _gather_ab(a_tab, b_tab, src_h[half], dst_gh[half])
            ms.append(_edge_mlp(g, rbf_h[half], w1r, msg_W2[l], msg_b2[l]))
        for half in range(2):
            parts.extend(_scatter_sum(ms[half], dst_ph[half], zero_rows))
        h = _update(parts[0], parts[1], parts[2], parts[3], h,
                    upd_W1[l], upd_b1[l], upd_W2[l], upd_b2[l])

    return _readout(h, won_W1, won_b1, won_W2, won_b2,
                    out_W1, out_b1, out_W2, out_b2)
